# Initial kernel scaffold; baseline (speedup 1.0000x reference)
import jax
import jax.numpy as jnp
from jax.experimental import pallas as pl


def kernel(x, edge_index, batch, W1, b1, W2, b2, W3, b3, Wl1, bl1, Wl2, bl2):
    raise NotImplementedError("write your pallas kernel here")


# speedup vs baseline: 19.3429x; 19.3429x over previous
import functools

import jax
import jax.numpy as jnp
from jax import lax
from jax.experimental import pallas as pl
from jax.experimental.pallas import tpu as pltpu
from jax.experimental.pallas import tpu_sc as plsc

N = 10000
G = 128
E = 320000

NC = 2
NS = 16
NW = NC * NS
CHUNK = 128
CPW = 80
EPW = CPW * CHUNK
EP = NW * EPW
RPT = 632
NP = RPT * NS

_MESH = plsc.VectorSubcoreMesh(core_axis_name="c", subcore_axis_name="s")


@functools.partial(
    pl.kernel,
    out_type=jax.ShapeDtypeStruct((NW, NP), jnp.float32),
    mesh=_MESH,
    scratch_types=[
        pltpu.VMEM((EPW,), jnp.int32),
        pltpu.VMEM((NP,), jnp.float32),
    ],
    compiler_params=pltpu.CompilerParams(
        needs_layout_passes=False, use_tc_tiling_on_sc=False),
)
def _deg_kernel(dstf_hbm, zerosn_hbm, out_hbm, idx_v, acc_v):
    c = lax.axis_index("c")
    s = lax.axis_index("s")
    wid = s * NC + c
    pltpu.sync_copy(dstf_hbm.at[wid], idx_v)
    pltpu.sync_copy(zerosn_hbm, acc_v)
    ones = jnp.ones((16,), jnp.float32)

    def body(i, carry):
        idx = idx_v[pl.ds(i * 16, 16)]
        plsc.addupdate_scatter(acc_v, [idx], ones)
        return carry

    lax.fori_loop(0, EPW // 16, body, 0)
    pltpu.sync_copy(acc_v, out_hbm.at[wid])


def _make_prop(d):
    @functools.partial(
        pl.kernel,
        out_type=jax.ShapeDtypeStruct((NC, NP, d), jnp.float32),
        mesh=_MESH,
        scratch_types=[
            pltpu.VMEM((CPW, CHUNK), jnp.int32),
            pltpu.VMEM((CPW, CHUNK), jnp.int32),
            pltpu.VMEM((CHUNK, d), jnp.float32),
            pltpu.VMEM((CHUNK, d), jnp.float32),
            pltpu.VMEM_SHARED((NP, d), jnp.float32),
            pltpu.SemaphoreType.DMA,
            pltpu.SemaphoreType.DMA,
        ],
        compiler_params=pltpu.CompilerParams(use_tc_tiling_on_sc=False),
    )
    def prop(h_hbm, srcp_hbm, dstp_hbm, zeros_hbm, out_hbm,
             src_v, dst_v, buf_a, buf_b, acc, sem_a, sem_b):
        c = lax.axis_index("c")
        s = lax.axis_index("s")
        wid = s * NC + c
        pltpu.sync_copy(zeros_hbm.at[pl.ds(s * RPT, RPT)],
                        acc.at[pl.ds(s * RPT, RPT)])
        pltpu.sync_copy(srcp_hbm.at[wid], src_v)
        pltpu.sync_copy(dstp_hbm.at[wid], dst_v)
        plsc.subcore_barrier()

        def body(t, carry):
            j = t * 2
            cp0 = pltpu.async_copy(h_hbm.at[src_v.at[j]], buf_a, sem_a)
            cp1 = pltpu.async_copy(h_hbm.at[src_v.at[j + 1]], buf_b, sem_b)
            cp0.wait()
            pltpu.sync_copy(buf_a, acc.at[dst_v.at[j]], add=True)
            cp1.wait()
            pltpu.sync_copy(buf_b, acc.at[dst_v.at[j + 1]], add=True)
            return carry

        lax.fori_loop(0, CPW // 2, body, 0)
        plsc.subcore_barrier()
        pltpu.sync_copy(acc.at[pl.ds(s * RPT, RPT)],
                        out_hbm.at[c, pl.ds(s * RPT, RPT)])

    return prop


_prop64 = _make_prop(64)
_prop32 = _make_prop(32)
_prop16 = _make_prop(16)


def _tc_first_body(x_ref, w_ref, degp_ref, h1p_ref, dis_ref):
    counts = jnp.sum(degp_ref[...], axis=0)
    dis = lax.rsqrt(counts[0:N] + 1.0)[:, None]
    dis_ref[...] = dis
    h = jnp.dot(x_ref[...], w_ref[...], preferred_element_type=jnp.float32)
    h1p_ref[...] = h * dis


def _tc_mid_body(sp_ref, hp_ref, dis_ref, b_ref, w_ref, out_ref):
    dis = dis_ref[...]
    a = dis * (sp_ref[0, 0:N, :] + sp_ref[1, 0:N, :] + hp_ref[...]) \
        + b_ref[...][None, :]
    a = jnp.maximum(a, 0.0)
    out_ref[...] = jnp.dot(a, w_ref[...],
                           preferred_element_type=jnp.float32) * dis


def _tc_final_body(sp_ref, hp_ref, dis_ref, b3_ref, batch_ref,
                   wl1_ref, bl1_ref, wl2_ref, bl2_ref, out_ref):
    feats = dis_ref[...] * (sp_ref[0, 0:N, :] + sp_ref[1, 0:N, :]
                            + hp_ref[...]) + b3_ref[...][None, :]
    gids = lax.broadcasted_iota(jnp.int32, (G, N), 0)
    mask = (batch_ref[...] == gids).astype(jnp.float32)
    pooled = jnp.dot(mask, feats, preferred_element_type=jnp.float32)
    cnt = jnp.sum(mask, axis=1, keepdims=True)
    mean = pooled / jnp.maximum(cnt, 1.0)
    h = jnp.dot(mean, wl1_ref[...], preferred_element_type=jnp.float32) \
        + bl1_ref[...][None, :]
    h = jnp.maximum(h, 0.0)
    o = jnp.dot(h, wl2_ref[...], preferred_element_type=jnp.float32) \
        + bl2_ref[...][None, :]
    out_ref[...] = jax.nn.sigmoid(o)


def _tc_first(x, w1, degp):
    return pl.pallas_call(
        _tc_first_body,
        out_shape=(jax.ShapeDtypeStruct((N, 64), jnp.float32),
                   jax.ShapeDtypeStruct((N, 1), jnp.float32)),
    )(x, w1, degp)


def _tc_mid(sp, hp, dis, b, w, d_out):
    return pl.pallas_call(
        _tc_mid_body,
        out_shape=jax.ShapeDtypeStruct((N, d_out), jnp.float32),
    )(sp, hp, dis, b, w)


def _tc_final(sp, hp, dis, b3, batch2d, wl1, bl1, wl2, bl2):
    return pl.pallas_call(
        _tc_final_body,
        out_shape=jax.ShapeDtypeStruct((G, 1), jnp.float32),
    )(sp, hp, dis, b3, batch2d, wl1, bl1, wl2, bl2)


def kernel(x, edge_index, batch, W1, b1, W2, b2, W3, b3, Wl1, bl1, Wl2, bl2):
    src = edge_index[0].astype(jnp.int32)
    dst = edge_index[1].astype(jnp.int32)
    pad = EP - E
    srcp = jnp.concatenate([src, jnp.zeros((pad,), jnp.int32)])
    srcp = srcp.reshape(NW, CPW, CHUNK)
    dstp = jnp.concatenate([dst, jnp.full((pad,), N, jnp.int32)])
    dstp = dstp.reshape(NW, CPW, CHUNK)
    dstf = dstp.reshape(NW, EPW)

    zerosn = jnp.zeros((NP,), jnp.float32)
    zeros64 = jnp.zeros((NP, 64), jnp.float32)
    zeros32 = jnp.zeros((NP, 32), jnp.float32)
    zeros16 = jnp.zeros((NP, 16), jnp.float32)
    batch2d = batch.astype(jnp.int32).reshape(1, N)

    degp = _deg_kernel(dstf, zerosn)
    h1p, dis = _tc_first(x, W1, degp)
    s1 = _prop64(h1p, srcp, dstp, zeros64)
    h2p = _tc_mid(s1, h1p, dis, b1, W2, 32)
    s2 = _prop32(h2p, srcp, dstp, zeros32)
    h3p = _tc_mid(s2, h2p, dis, b2, W3, 16)
    s3 = _prop16(h3p, srcp, dstp, zeros16)
    return _tc_final(s3, h3p, dis, b3, batch2d, Wl1, bl1, Wl2, bl2)


# speedup vs baseline: 21.6197x; 1.1177x over previous
import functools

import jax
import jax.numpy as jnp
from jax import lax
from jax.experimental import pallas as pl
from jax.experimental.pallas import tpu as pltpu
from jax.experimental.pallas import tpu_sc as plsc

N = 10000
G = 128
E = 320000

NC = 2
NS = 16
NW = NC * NS
CHUNK = 128
CPW = 80
EPW = CPW * CHUNK
EP = NW * EPW
RPT = 632
NP = RPT * NS

_MESH = plsc.VectorSubcoreMesh(core_axis_name="c", subcore_axis_name="s")


@functools.partial(
    pl.kernel,
    out_type=jax.ShapeDtypeStruct((NW, NP), jnp.float32),
    mesh=_MESH,
    scratch_types=[
        pltpu.VMEM((EPW,), jnp.int32),
        pltpu.VMEM((NP,), jnp.float32),
    ],
    compiler_params=pltpu.CompilerParams(
        needs_layout_passes=False, use_tc_tiling_on_sc=False),
)
def _deg_kernel(dstf_hbm, zerosn_hbm, out_hbm, idx_v, acc_v):
    c = lax.axis_index("c")
    s = lax.axis_index("s")
    wid = s * NC + c
    pltpu.sync_copy(dstf_hbm.at[wid], idx_v)
    pltpu.sync_copy(zerosn_hbm, acc_v)
    ones = jnp.ones((16,), jnp.float32)

    def body(i, carry):
        idx = idx_v[pl.ds(i * 16, 16)]
        plsc.addupdate_scatter(acc_v, [idx], ones)
        return carry

    lax.fori_loop(0, EPW // 16, body, 0)
    pltpu.sync_copy(acc_v, out_hbm.at[wid])


NBUF = 4


def _make_prop(d):
    @functools.partial(
        pl.kernel,
        out_type=jax.ShapeDtypeStruct((NC, NP, d), jnp.float32),
        mesh=_MESH,
        scratch_types=[
            pltpu.VMEM((CPW, CHUNK), jnp.int32),
            pltpu.VMEM((CPW, CHUNK), jnp.int32),
            [pltpu.VMEM((CHUNK, d), jnp.float32)] * NBUF,
            pltpu.VMEM_SHARED((NP, d), jnp.float32),
            [pltpu.SemaphoreType.DMA] * NBUF,
            [pltpu.SemaphoreType.DMA] * NBUF,
        ],
        compiler_params=pltpu.CompilerParams(use_tc_tiling_on_sc=False),
    )
    def prop(h_hbm, srcp_hbm, dstp_hbm, zeros_hbm, out_hbm,
             src_v, dst_v, bufs, acc, gsem, ssem):
        c = lax.axis_index("c")
        s = lax.axis_index("s")
        wid = s * NC + c
        pltpu.sync_copy(zeros_hbm.at[pl.ds(s * RPT, RPT)],
                        acc.at[pl.ds(s * RPT, RPT)])
        pltpu.sync_copy(srcp_hbm.at[wid], src_v)
        pltpu.sync_copy(dstp_hbm.at[wid], dst_v)
        plsc.subcore_barrier()

        def gather(j, b):
            pltpu.async_copy(h_hbm.at[src_v.at[j]], bufs[b], gsem[b])

        def gather_wait(j, b):
            pltpu.make_async_copy(h_hbm.at[src_v.at[j]], bufs[b],
                                  gsem[b]).wait()

        def scat(j, b):
            pltpu.async_copy(bufs[b], acc.at[dst_v.at[j]], ssem[b], add=True)

        def scat_wait(j, b):
            pltpu.make_async_copy(bufs[b], acc.at[dst_v.at[j]],
                                  ssem[b]).wait()

        for b in range(NBUF):
            gather(b, b)

        def round_body(i, carry):
            t0 = i * NBUF
            for b in range(NBUF):
                gather_wait(t0 + b, b)
                scat(t0 + b, b)
            for b in range(NBUF):
                scat_wait(t0 + b, b)
                gather(t0 + NBUF + b, b)
            return carry

        lax.fori_loop(0, CPW // NBUF - 1, round_body, 0)
        t0 = CPW - NBUF
        for b in range(NBUF):
            gather_wait(t0 + b, b)
            scat(t0 + b, b)
        for b in range(NBUF):
            scat_wait(t0 + b, b)
        plsc.subcore_barrier()
        pltpu.sync_copy(acc.at[pl.ds(s * RPT, RPT)],
                        out_hbm.at[c, pl.ds(s * RPT, RPT)])

    return prop


_prop64 = _make_prop(64)
_prop32 = _make_prop(32)
_prop16 = _make_prop(16)


def _tc_first_body(x_ref, w_ref, degp_ref, h1p_ref, dis_ref):
    counts = jnp.sum(degp_ref[...], axis=0)
    dis = lax.rsqrt(counts[0:N] + 1.0)[:, None]
    dis_ref[...] = dis
    h = jnp.dot(x_ref[...], w_ref[...], preferred_element_type=jnp.float32)
    h1p_ref[...] = h * dis


def _tc_mid_body(sp_ref, hp_ref, dis_ref, b_ref, w_ref, out_ref):
    dis = dis_ref[...]
    a = dis * (sp_ref[0, 0:N, :] + sp_ref[1, 0:N, :] + hp_ref[...]) \
        + b_ref[...][None, :]
    a = jnp.maximum(a, 0.0)
    out_ref[...] = jnp.dot(a, w_ref[...],
                           preferred_element_type=jnp.float32) * dis


def _tc_final_body(sp_ref, hp_ref, dis_ref, b3_ref, batch_ref,
                   wl1_ref, bl1_ref, wl2_ref, bl2_ref, out_ref):
    feats = dis_ref[...] * (sp_ref[0, 0:N, :] + sp_ref[1, 0:N, :]
                            + hp_ref[...]) + b3_ref[...][None, :]
    gids = lax.broadcasted_iota(jnp.int32, (G, N), 0)
    mask = (batch_ref[...] == gids).astype(jnp.float32)
    pooled = jnp.dot(mask, feats, preferred_element_type=jnp.float32)
    cnt = jnp.sum(mask, axis=1, keepdims=True)
    mean = pooled / jnp.maximum(cnt, 1.0)
    h = jnp.dot(mean, wl1_ref[...], preferred_element_type=jnp.float32) \
        + bl1_ref[...][None, :]
    h = jnp.maximum(h, 0.0)
    o = jnp.dot(h, wl2_ref[...], preferred_element_type=jnp.float32) \
        + bl2_ref[...][None, :]
    out_ref[...] = jax.nn.sigmoid(o)


def _tc_first(x, w1, degp):
    return pl.pallas_call(
        _tc_first_body,
        out_shape=(jax.ShapeDtypeStruct((N, 64), jnp.float32),
                   jax.ShapeDtypeStruct((N, 1), jnp.float32)),
    )(x, w1, degp)


def _tc_mid(sp, hp, dis, b, w, d_out):
    return pl.pallas_call(
        _tc_mid_body,
        out_shape=jax.ShapeDtypeStruct((N, d_out), jnp.float32),
    )(sp, hp, dis, b, w)


def _tc_final(sp, hp, dis, b3, batch2d, wl1, bl1, wl2, bl2):
    return pl.pallas_call(
        _tc_final_body,
        out_shape=jax.ShapeDtypeStruct((G, 1), jnp.float32),
    )(sp, hp, dis, b3, batch2d, wl1, bl1, wl2, bl2)


def kernel(x, edge_index, batch, W1, b1, W2, b2, W3, b3, Wl1, bl1, Wl2, bl2):
    src = edge_index[0].astype(jnp.int32)
    dst = edge_index[1].astype(jnp.int32)
    pad = EP - E
    srcp = jnp.concatenate([src, jnp.zeros((pad,), jnp.int32)])
    srcp = srcp.reshape(NW, CPW, CHUNK)
    dstp = jnp.concatenate([dst, jnp.full((pad,), N, jnp.int32)])
    dstp = dstp.reshape(NW, CPW, CHUNK)
    dstf = dstp.reshape(NW, EPW)

    zerosn = jnp.zeros((NP,), jnp.float32)
    zeros64 = jnp.zeros((NP, 64), jnp.float32)
    zeros32 = jnp.zeros((NP, 32), jnp.float32)
    zeros16 = jnp.zeros((NP, 16), jnp.float32)
    batch2d = batch.astype(jnp.int32).reshape(1, N)

    degp = _deg_kernel(dstf, zerosn)
    h1p, dis = _tc_first(x, W1, degp)
    s1 = _prop64(h1p, srcp, dstp, zeros64)
    h2p = _tc_mid(s1, h1p, dis, b1, W2, 32)
    s2 = _prop32(h2p, srcp, dstp, zeros32)
    h3p = _tc_mid(s2, h2p, dis, b2, W3, 16)
    s3 = _prop16(h3p, srcp, dstp, zeros16)
    return _tc_final(s3, h3p, dis, b3, batch2d, Wl1, bl1, Wl2, bl2)
